# baseline (speedup 1.0000x reference)
import math

import numpy as np
import jax
import jax.numpy as jnp
from jax import lax
from jax.experimental import pallas as pl
from jax.experimental.pallas import tpu as pltpu

_S, _D = 2048, 768
_H, _HD, _FACTOR = 12, 64, 5
_SK = min(_FACTOR * math.ceil(math.log(_S)), _S)
_NT = min(_FACTOR * math.ceil(math.log(_S)), _S)
_QB = 512
_NQB = _S // _QB


def _threefry2x32(k1, k2, x0, x1):
    rot0, rot1 = (13, 15, 26, 6), (17, 29, 16, 24)
    ks = (np.uint32(k1), np.uint32(k2),
          np.uint32(k1) ^ np.uint32(k2) ^ np.uint32(0x1BD11BDA))
    x0 = (x0 + ks[0]).astype(np.uint32)
    x1 = (x1 + ks[1]).astype(np.uint32)
    sched = ((rot0, ks[1], ks[2], 1), (rot1, ks[2], ks[0], 2),
             (rot0, ks[0], ks[1], 3), (rot1, ks[1], ks[2], 4),
             (rot0, ks[2], ks[0], 5))
    for rots, a, b, i in sched:
        for r in rots:
            x0 = (x0 + x1).astype(np.uint32)
            x1 = ((x1 << np.uint32(r)) | (x1 >> np.uint32(32 - r))).astype(np.uint32)
            x1 = x0 ^ x1
        x0 = (x0 + a).astype(np.uint32)
        x1 = (x1 + b + np.uint32(i)).astype(np.uint32)
    return x0, x1


def _build_count() -> np.ndarray:
    k1, k2 = np.uint32(0), np.uint32(42)
    b1, b2 = _threefry2x32(k1, k2, np.zeros(2, np.uint32),
                           np.arange(2, dtype=np.uint32))
    n = _S * _SK
    zeros = np.zeros(n, np.uint32)
    cnts = np.arange(n, dtype=np.uint32)
    hi1, hi2 = _threefry2x32(b1[0], b2[0], zeros, cnts)
    lo1, lo2 = _threefry2x32(b1[1], b2[1], zeros, cnts)
    higher_bits, lower_bits = hi1 ^ hi2, lo1 ^ lo2
    span = np.uint32(_S)
    mult = np.uint32((2 ** 16) % _S)
    mult = np.uint32((int(mult) * int(mult)) % _S)
    off = ((higher_bits % span) * mult + lower_bits % span) % span
    idx = off.astype(np.int32).reshape(_S, _SK)
    cnt = np.zeros((_S, _S), np.int8)
    np.add.at(cnt, (np.arange(_S)[:, None], idx), 1)
    return cnt


_COUNT = _build_count()
_NC = 64
_DN_T = (((1,), (1,)), ((), ()))
_DN_N = (((1,), (0,)), ((), ()))


def _proj_body(x_ref, wq_ref, bq_ref, wk_ref, bk_ref, wv_ref, bv_ref,
               q_ref, k_ref, v_ref, qb_ref, kb_ref):
    x = x_ref[...]
    q = lax.dot_general(x, wq_ref[...], _DN_T,
                        preferred_element_type=jnp.float32) + bq_ref[...]
    k = lax.dot_general(x, wk_ref[...], _DN_T,
                        preferred_element_type=jnp.float32) + bk_ref[...]
    v = lax.dot_general(x, wv_ref[...], _DN_T,
                        preferred_element_type=jnp.float32) + bv_ref[...]
    for h in range(_H):
        sl = slice(h * _HD, (h + 1) * _HD)
        q_ref[h] = q[:, sl]
        k_ref[h] = k[:, sl]
        v_ref[h] = v[:, sl]
        qb_ref[h] = q[:, sl].astype(jnp.bfloat16)
        kb_ref[h] = k[:, sl].astype(jnp.bfloat16)


def _qkv(x, wq, bq, wk, bk, wv, bv):
    full_w = pl.BlockSpec((_D, _D), lambda i: (0, 0))
    full_b = pl.BlockSpec((_D,), lambda i: (0,))
    out_blk = pl.BlockSpec((_H, _QB, _HD), lambda i: (0, i, 0))
    out = jax.ShapeDtypeStruct((_H, _S, _HD), jnp.float32)
    outb = jax.ShapeDtypeStruct((_H, _S, _HD), jnp.bfloat16)
    return pl.pallas_call(
        _proj_body,
        grid=(_NQB,),
        in_specs=[pl.BlockSpec((_QB, _D), lambda i: (i, 0)),
                  full_w, full_b, full_w, full_b, full_w, full_b],
        out_specs=[out_blk, out_blk, out_blk, out_blk, out_blk],
        out_shape=[out, out, out, outb, outb],
    )(x, wq, bq, wk, bk, wv, bv)


def _m_body(q_ref, k_ref, cnt_ref, m_ref):
    cntf = cnt_ref[...].astype(jnp.float32)
    sel = cntf > 0.0
    for h in range(_H):
        s = lax.dot_general(q_ref[h], k_ref[h], _DN_T,
                            preferred_element_type=jnp.float32)
        msum = jnp.sum(s * cntf, axis=1) * (1.0 / _S)
        mmax = jnp.max(jnp.where(sel, s, -1e30), axis=1)
        m_ref[h, :] = mmax - msum


def _m_scores(q, k, cnt):
    return pl.pallas_call(
        _m_body,
        grid=(_NQB,),
        in_specs=[
            pl.BlockSpec((_H, _QB, _HD), lambda i: (0, i, 0)),
            pl.BlockSpec((_H, _S, _HD), lambda i: (0, 0, 0)),
            pl.BlockSpec((_QB, _S), lambda i: (i, 0)),
        ],
        out_specs=pl.BlockSpec((_H, _QB), lambda i: (0, i)),
        out_shape=jax.ShapeDtypeStruct((_H, _S), jnp.float32),
    )(q, k, cnt)


def _topk_body(m_ref, idx_ref):
    iota = lax.broadcasted_iota(jnp.int32, (_H, _S), 1)

    def step(j, vals):
        mx = jnp.max(vals, axis=1, keepdims=True)
        idx = jnp.min(jnp.where(vals >= mx, iota, 2 * _S), axis=1)
        idx_ref[pl.ds(j, 1), :] = idx[None, :]
        return jnp.where(iota == idx[:, None], -jnp.inf, vals)

    lax.fori_loop(0, _NC, step, m_ref[...])


def _topk(m):
    return pl.pallas_call(
        _topk_body,
        out_shape=jax.ShapeDtypeStruct((_NC, _H), jnp.int32),
    )(m)


def _rescore_body(q_ref, k_ref, cnt_ref, i64s_ref, i64v_ref, mt_ref,
                  qc_ref, crow_ref, sem):
    h = pl.program_id(0)
    copies = []
    for j in range(_NC):
        idx = i64s_ref[h, 0, j]
        qc_ref[pl.ds(j, 1), :] = q_ref[0, pl.ds(idx, 1), :]
        copies.append(pltpu.make_async_copy(
            cnt_ref.at[pl.ds(idx, 1)], crow_ref.at[pl.ds(j, 1)], sem))
    for c in copies:
        c.start()
    for c in copies:
        c.wait()
    s = lax.dot_general(qc_ref[...], k_ref[0], _DN_T,
                        preferred_element_type=jnp.float32)
    cntf = crow_ref[:, 0, :].astype(jnp.float32)
    msum = jnp.sum(s * cntf, axis=1) * (1.0 / _S)
    mmax = jnp.max(jnp.where(cntf > 0.0, s, -1e30), axis=1)
    m64 = (mmax - msum)[None, :]
    idxv = i64v_ref[0]
    iota_c = lax.broadcasted_iota(jnp.int32, (1, _NC), 1)
    iota_t = lax.broadcasted_iota(jnp.int32, (1, _NT), 1)

    def step(j, carry):
        vals, out = carry
        mx = jnp.max(vals, axis=1, keepdims=True)
        pos = jnp.min(jnp.where(vals >= mx, iota_c, 2 * _NC),
                      axis=1, keepdims=True)
        val = jnp.sum(jnp.where(iota_c == pos, idxv, 0),
                      axis=1, keepdims=True)
        out = jnp.where(iota_t == j, val, out)
        vals = jnp.where(iota_c == pos, -jnp.inf, vals)
        return vals, out

    _, out = lax.fori_loop(0, _NT, step,
                           (m64, jnp.zeros((1, _NT), jnp.int32)))
    mt_ref[0] = out


def _rescore(q, k, cnt, i64):
    return pl.pallas_call(
        _rescore_body,
        grid=(_H,),
        in_specs=[
            pl.BlockSpec((1, _S, _HD), lambda h: (h, 0, 0)),
            pl.BlockSpec((1, _S, _HD), lambda h: (h, 0, 0)),
            pl.BlockSpec(memory_space=pl.ANY),
            pl.BlockSpec(memory_space=pltpu.SMEM),
            pl.BlockSpec((1, 1, _NC), lambda h: (h, 0, 0)),
        ],
        out_specs=pl.BlockSpec((1, 1, _NT), lambda h: (h, 0, 0)),
        out_shape=jax.ShapeDtypeStruct((_H, 1, _NT), jnp.int32),
        scratch_shapes=[
            pltpu.VMEM((_NC, _HD), jnp.float32),
            pltpu.VMEM((_NC, 1, _S), jnp.int8),
            pltpu.SemaphoreType.DMA,
        ],
    )(q, k, cnt, i64, i64)


def _attn_body(q_ref, k_ref, v_ref, mt_ref, wo_ref, bo_ref,
               o_ref, qr_ref, d_scr, base_scr):
    h = pl.program_id(0)
    for j in range(_NT):
        idx = mt_ref[h, 0, j]
        qr_ref[pl.ds(j, 1), :] = q_ref[0, pl.ds(idx, 1), :]
    s = lax.dot_general(qr_ref[...], k_ref[0], _DN_T,
                        preferred_element_type=jnp.float32) * (1.0 / math.sqrt(_HD))
    mx = jnp.max(s, axis=1, keepdims=True)
    e = jnp.exp(s - mx)
    attn = e / jnp.sum(e, axis=1, keepdims=True)
    upd = lax.dot_general(attn, v_ref[0], _DN_N,
                          preferred_element_type=jnp.float32)
    vmean = jnp.mean(v_ref[0], axis=0)
    d_scr[h] = lax.dot_general(upd - vmean[None, :], wo_ref[0], _DN_T,
                               preferred_element_type=jnp.float32)
    bvec = lax.dot_general(vmean[None, :], wo_ref[0], _DN_T,
                           preferred_element_type=jnp.float32)

    @pl.when(h == 0)
    def _():
        base_scr[...] = bo_ref[...][None, :] + bvec

    @pl.when(h != 0)
    def _():
        base_scr[...] = base_scr[...] + bvec

    @pl.when(h == _H - 1)
    def _():
        o_ref[...] = jnp.broadcast_to(base_scr[...], (_S, _D))
        for hh in range(_H):
            for j in range(_NT):
                idx = mt_ref[hh, 0, j]
                o_ref[pl.ds(idx, 1), :] = (o_ref[pl.ds(idx, 1), :]
                                           + d_scr[hh, pl.ds(j, 1), :])


def _sparse_attn(q, k, v, mt, wo3, bo):
    col = pl.BlockSpec((1, _S, _HD), lambda h: (h, 0, 0))
    return pl.pallas_call(
        _attn_body,
        grid=(_H,),
        in_specs=[
            col, col, col,
            pl.BlockSpec(memory_space=pltpu.SMEM),
            pl.BlockSpec((1, _D, _HD), lambda h: (h, 0, 0)),
            pl.BlockSpec((_D,), lambda h: (0,)),
        ],
        out_specs=pl.BlockSpec((_S, _D), lambda h: (0, 0)),
        out_shape=jax.ShapeDtypeStruct((_S, _D), jnp.float32),
        scratch_shapes=[
            pltpu.VMEM((_NT, _HD), jnp.float32),
            pltpu.VMEM((_H, _NT, _D), jnp.float32),
            pltpu.VMEM((1, _D), jnp.float32),
        ],
    )(q, k, v, mt, wo3, bo)


def kernel(hidden_states, Wq, bq, Wk, bk, Wv, bv, Wo, bo):
    x = hidden_states[0]
    cnt = jnp.asarray(_COUNT)
    q, k, v, qb, kb = _qkv(x, Wq, bq, Wk, bk, Wv, bv)
    m = _m_scores(qb, kb, cnt)
    i64 = _topk(m).T.reshape(_H, 1, _NC)
    mt = _rescore(q, k, cnt.reshape(_S, 1, _S), i64)
    wo3 = Wo.reshape(_D, _H, _HD).transpose(1, 0, 2)
    out = _sparse_attn(q, k, v, mt, wo3, bo)
    return out[None]


# speedup vs baseline: 2.5791x; 2.5791x over previous
import math

import numpy as np
import jax
import jax.numpy as jnp
from jax import lax
from jax.experimental import pallas as pl
from jax.experimental.pallas import tpu as pltpu

_S, _D = 2048, 768
_H, _HD, _FACTOR = 12, 64, 5
_SK = min(_FACTOR * math.ceil(math.log(_S)), _S)
_NT = min(_FACTOR * math.ceil(math.log(_S)), _S)
_QB = 512
_NQB = _S // _QB


def _threefry2x32(k1, k2, x0, x1):
    rot0, rot1 = (13, 15, 26, 6), (17, 29, 16, 24)
    ks = (np.uint32(k1), np.uint32(k2),
          np.uint32(k1) ^ np.uint32(k2) ^ np.uint32(0x1BD11BDA))
    x0 = (x0 + ks[0]).astype(np.uint32)
    x1 = (x1 + ks[1]).astype(np.uint32)
    sched = ((rot0, ks[1], ks[2], 1), (rot1, ks[2], ks[0], 2),
             (rot0, ks[0], ks[1], 3), (rot1, ks[1], ks[2], 4),
             (rot0, ks[2], ks[0], 5))
    for rots, a, b, i in sched:
        for r in rots:
            x0 = (x0 + x1).astype(np.uint32)
            x1 = ((x1 << np.uint32(r)) | (x1 >> np.uint32(32 - r))).astype(np.uint32)
            x1 = x0 ^ x1
        x0 = (x0 + a).astype(np.uint32)
        x1 = (x1 + b + np.uint32(i)).astype(np.uint32)
    return x0, x1


def _build_count() -> np.ndarray:
    k1, k2 = np.uint32(0), np.uint32(42)
    b1, b2 = _threefry2x32(k1, k2, np.zeros(2, np.uint32),
                           np.arange(2, dtype=np.uint32))
    n = _S * _SK
    zeros = np.zeros(n, np.uint32)
    cnts = np.arange(n, dtype=np.uint32)
    hi1, hi2 = _threefry2x32(b1[0], b2[0], zeros, cnts)
    lo1, lo2 = _threefry2x32(b1[1], b2[1], zeros, cnts)
    higher_bits, lower_bits = hi1 ^ hi2, lo1 ^ lo2
    span = np.uint32(_S)
    mult = np.uint32((2 ** 16) % _S)
    mult = np.uint32((int(mult) * int(mult)) % _S)
    off = ((higher_bits % span) * mult + lower_bits % span) % span
    idx = off.astype(np.int32).reshape(_S, _SK)
    cnt = np.zeros((_S, _S), np.int8)
    np.add.at(cnt, (np.arange(_S)[:, None], idx), 1)
    return cnt


_COUNT = _build_count()
_DN_T = (((1,), (1,)), ((), ()))
_DN_N = (((1,), (0,)), ((), ()))


def _proj_body(x_ref, wq_ref, bq_ref, wk_ref, bk_ref, wv_ref, bv_ref,
               q_ref, k_ref, v_ref):
    x = x_ref[...]
    q = lax.dot_general(x, wq_ref[...], _DN_T,
                        preferred_element_type=jnp.float32) + bq_ref[...]
    k = lax.dot_general(x, wk_ref[...], _DN_T,
                        preferred_element_type=jnp.float32) + bk_ref[...]
    v = lax.dot_general(x, wv_ref[...], _DN_T,
                        preferred_element_type=jnp.float32) + bv_ref[...]
    for h in range(_H):
        sl = slice(h * _HD, (h + 1) * _HD)
        q_ref[h] = q[:, sl]
        k_ref[h] = k[:, sl]
        v_ref[h] = v[:, sl]


def _qkv(x, wq, bq, wk, bk, wv, bv):
    full_w = pl.BlockSpec((_D, _D), lambda i: (0, 0))
    full_b = pl.BlockSpec((_D,), lambda i: (0,))
    out_blk = pl.BlockSpec((_H, _QB, _HD), lambda i: (0, i, 0))
    out = jax.ShapeDtypeStruct((_H, _S, _HD), jnp.float32)
    return pl.pallas_call(
        _proj_body,
        grid=(_NQB,),
        in_specs=[pl.BlockSpec((_QB, _D), lambda i: (i, 0)),
                  full_w, full_b, full_w, full_b, full_w, full_b],
        out_specs=[out_blk, out_blk, out_blk],
        out_shape=[out, out, out],
    )(x, wq, bq, wk, bk, wv, bv)


def _m_body(q_ref, k_ref, cnt_ref, m_ref):
    cntf = cnt_ref[...].astype(jnp.float32)
    sel = cntf > 0.0
    for h in range(_H):
        s = lax.dot_general(q_ref[h], k_ref[h], _DN_T,
                            preferred_element_type=jnp.float32)
        msum = jnp.sum(s * cntf, axis=1) * (1.0 / _S)
        mmax = jnp.max(jnp.where(sel, s, -1e30), axis=1)
        m_ref[h, :] = mmax - msum


def _m_scores(q, k, cnt):
    return pl.pallas_call(
        _m_body,
        grid=(_NQB,),
        in_specs=[
            pl.BlockSpec((_H, _QB, _HD), lambda i: (0, i, 0)),
            pl.BlockSpec((_H, _S, _HD), lambda i: (0, 0, 0)),
            pl.BlockSpec((_QB, _S), lambda i: (i, 0)),
        ],
        out_specs=pl.BlockSpec((_H, _QB), lambda i: (0, i)),
        out_shape=jax.ShapeDtypeStruct((_H, _S), jnp.float32),
    )(q, k, cnt)


def _topk_body(m_ref, idx_ref):
    iota = lax.broadcasted_iota(jnp.int32, (_H, _S), 1)

    def step(j, vals):
        mx = jnp.max(vals, axis=1, keepdims=True)
        idx = jnp.min(jnp.where(vals >= mx, iota, 2 * _S), axis=1)
        idx_ref[pl.ds(j, 1), :] = idx[None, :]
        return jnp.where(iota == idx[:, None], -jnp.inf, vals)

    lax.fori_loop(0, _NT, step, m_ref[...])


def _topk(m):
    return pl.pallas_call(
        _topk_body,
        out_shape=jax.ShapeDtypeStruct((_NT, _H), jnp.int32),
    )(m)


def _attn_body(q_ref, k_ref, v_ref, mt_ref, wo_ref, bo_ref,
               o_ref, qr_ref, d_scr, base_scr):
    h = pl.program_id(0)
    for j in range(_NT):
        idx = mt_ref[h, 0, j]
        qr_ref[pl.ds(j, 1), :] = q_ref[0, pl.ds(idx, 1), :]
    s = lax.dot_general(qr_ref[...], k_ref[0], _DN_T,
                        preferred_element_type=jnp.float32) * (1.0 / math.sqrt(_HD))
    mx = jnp.max(s, axis=1, keepdims=True)
    e = jnp.exp(s - mx)
    attn = e / jnp.sum(e, axis=1, keepdims=True)
    upd = lax.dot_general(attn, v_ref[0], _DN_N,
                          preferred_element_type=jnp.float32)
    vmean = jnp.mean(v_ref[0], axis=0)
    d_scr[h] = lax.dot_general(upd - vmean[None, :], wo_ref[0], _DN_T,
                               preferred_element_type=jnp.float32)
    bvec = lax.dot_general(vmean[None, :], wo_ref[0], _DN_T,
                           preferred_element_type=jnp.float32)

    @pl.when(h == 0)
    def _():
        base_scr[...] = bo_ref[...][None, :] + bvec

    @pl.when(h != 0)
    def _():
        base_scr[...] = base_scr[...] + bvec

    @pl.when(h == _H - 1)
    def _():
        o_ref[...] = jnp.broadcast_to(base_scr[...], (_S, _D))
        for hh in range(_H):
            for j in range(_NT):
                idx = mt_ref[hh, 0, j]
                o_ref[pl.ds(idx, 1), :] = (o_ref[pl.ds(idx, 1), :]
                                           + d_scr[hh, pl.ds(j, 1), :])


def _sparse_attn(q, k, v, mt, wo3, bo):
    col = pl.BlockSpec((1, _S, _HD), lambda h: (h, 0, 0))
    return pl.pallas_call(
        _attn_body,
        grid=(_H,),
        in_specs=[
            col, col, col,
            pl.BlockSpec(memory_space=pltpu.SMEM),
            pl.BlockSpec((1, _D, _HD), lambda h: (h, 0, 0)),
            pl.BlockSpec((_D,), lambda h: (0,)),
        ],
        out_specs=pl.BlockSpec((_S, _D), lambda h: (0, 0)),
        out_shape=jax.ShapeDtypeStruct((_S, _D), jnp.float32),
        scratch_shapes=[
            pltpu.VMEM((_NT, _HD), jnp.float32),
            pltpu.VMEM((_H, _NT, _D), jnp.float32),
            pltpu.VMEM((1, _D), jnp.float32),
        ],
    )(q, k, v, mt, wo3, bo)


def kernel(hidden_states, Wq, bq, Wk, bk, Wv, bv, Wo, bo):
    x = hidden_states[0]
    q, k, v = _qkv(x, Wq, bq, Wk, bk, Wv, bv)
    m = _m_scores(q, k, jnp.asarray(_COUNT))
    mt = _topk(m).T.reshape(_H, 1, _NT)
    wo3 = Wo.reshape(_D, _H, _HD).transpose(1, 0, 2)
    out = _sparse_attn(q, k, v, mt, wo3, bo)
    return out[None]
